# Initial kernel scaffold; baseline (speedup 1.0000x reference)
import jax
import jax.numpy as jnp
from jax.experimental import pallas as pl


def kernel(x, W_enc, b_enc, W_dec, b_dec):
    raise NotImplementedError("write your pallas kernel here")


# speedup vs baseline: 11.6748x; 11.6748x over previous
import functools

import jax
import jax.numpy as jnp
from jax import lax
from jax.experimental import pallas as pl
from jax.experimental.pallas import tpu as pltpu

K = 128
_MM_BF16 = True


def _dot(a, b):
    if _MM_BF16:
        a = a.astype(jnp.bfloat16)
        b = b.astype(jnp.bfloat16)
    return lax.dot_general(
        a, b,
        dimension_numbers=(((1,), (1,)), ((), ())),
        preferred_element_type=jnp.float32,
    )


def _encode_body(x_ref, w_ref, benc_ref, bdec_ref, out_ref):
    xc = x_ref[...] - bdec_ref[...]
    acc = _dot(xc, w_ref[...])
    out_ref[...] = jnp.maximum(acc + benc_ref[...], 0.0)


def _select_body(p_ref, acts_ref, *, k):
    v = p_ref[...]
    u = lax.bitcast_convert_type(v, jnp.int32)
    rb = v.shape[0]

    def body(i, prefix):
        cand = prefix | (1 << (30 - i))
        cnt = jnp.sum((u >= cand).astype(jnp.int32), axis=1, keepdims=True)
        return jnp.where(cnt >= k, cand, prefix)

    thr = lax.fori_loop(0, 31, body, jnp.zeros((rb, 1), jnp.int32))
    acts_ref[...] = jnp.where(u >= thr, v, 0.0)


def _decode_body(acts_ref, w_ref, bdec_ref, x_ref, xrec_ref, l2_ref, *, steps):
    i = pl.program_id(0)
    part = _dot(acts_ref[...], w_ref[...])

    @pl.when(i == 0)
    def _init():
        xrec_ref[...] = part

    @pl.when(i > 0)
    def _acc():
        xrec_ref[...] = xrec_ref[...] + part

    @pl.when(i == steps - 1)
    def _fin():
        xr = xrec_ref[...] + bdec_ref[...]
        xrec_ref[...] = xr
        d = xr - x_ref[...]
        l2_ref[...] = (jnp.sum(d * d) / d.shape[0])[None, None]


def kernel(x, W_enc, b_enc, W_dec, b_dec):
    B, A = x.shape
    D = W_enc.shape[0]
    DT = min(2048, D)
    steps = D // DT
    benc2 = b_enc.reshape(1, D)
    bdec2 = b_dec.reshape(1, A)

    post = pl.pallas_call(
        _encode_body,
        grid=(steps,),
        in_specs=[
            pl.BlockSpec((B, A), lambda i: (0, 0)),
            pl.BlockSpec((DT, A), lambda i: (i, 0)),
            pl.BlockSpec((1, DT), lambda i: (0, i)),
            pl.BlockSpec((1, A), lambda i: (0, 0)),
        ],
        out_specs=pl.BlockSpec((B, DT), lambda i: (0, i)),
        out_shape=jax.ShapeDtypeStruct((B, D), jnp.float32),
        compiler_params=pltpu.CompilerParams(
            dimension_semantics=("arbitrary",),
        ),
    )(x, W_enc, benc2, bdec2)

    RB = min(32, B)
    acts = pl.pallas_call(
        functools.partial(_select_body, k=K),
        grid=(B // RB,),
        in_specs=[pl.BlockSpec((RB, D), lambda i: (i, 0))],
        out_specs=pl.BlockSpec((RB, D), lambda i: (i, 0)),
        out_shape=jax.ShapeDtypeStruct((B, D), jnp.float32),
        compiler_params=pltpu.CompilerParams(
            dimension_semantics=("arbitrary",),
        ),
    )(post)

    xrec, l2 = pl.pallas_call(
        functools.partial(_decode_body, steps=steps),
        grid=(steps,),
        in_specs=[
            pl.BlockSpec((B, DT), lambda i: (0, i)),
            pl.BlockSpec((A, DT), lambda i: (0, i)),
            pl.BlockSpec((1, A), lambda i: (0, 0)),
            pl.BlockSpec((B, A), lambda i: (0, 0)),
        ],
        out_specs=[
            pl.BlockSpec((B, A), lambda i: (0, 0)),
            pl.BlockSpec((1, 1), lambda i: (0, 0)),
        ],
        out_shape=[
            jax.ShapeDtypeStruct((B, A), jnp.float32),
            jax.ShapeDtypeStruct((1, 1), jnp.float32),
        ],
        compiler_params=pltpu.CompilerParams(
            dimension_semantics=("arbitrary",),
        ),
    )(acts, W_dec, bdec2, x)

    l2s = l2[0, 0]
    l1 = jnp.asarray(0.0, dtype=jnp.float32)
    loss = l2s + l1
    return (loss, xrec, acts, l2s, l1)
